# Initial kernel scaffold; baseline (speedup 1.0000x reference)
import jax
import jax.numpy as jnp
from jax.experimental import pallas as pl


def kernel(x, edge_index, W1, b1, W2, b2):
    raise NotImplementedError("write your pallas kernel here")


# speedup vs baseline: 13.2707x; 13.2707x over previous
import functools

import jax
import jax.numpy as jnp
from jax.experimental import pallas as pl
from jax.experimental.pallas import tpu as pltpu

_N = 100
_NP = 128
_E = 3200


def _gcn_tc_kernel(src_ref, dst_ref, x_ref, w1_ref, b1_ref, w2_ref, b2_ref,
                   out_ref):
    f32 = jnp.float32
    hi = jax.lax.Precision.HIGHEST

    node_iota = jax.lax.broadcasted_iota(jnp.int32, (_E, _NP), 1)
    D = (dst_ref[:] == node_iota).astype(f32)
    S = (src_ref[:] == node_iota).astype(f32)

    eye = (jax.lax.broadcasted_iota(jnp.int32, (_NP, _NP), 0)
           == jax.lax.broadcasted_iota(jnp.int32, (_NP, _NP), 1)).astype(f32)
    adj = jax.lax.dot_general(D, S, (((0,), (0,)), ((), ())),
                              preferred_element_type=f32) + eye

    deg = jnp.sum(D, axis=0, keepdims=True) + 1.0
    dinv = jax.lax.rsqrt(deg)
    dmat = eye * dinv
    a = jnp.dot(jnp.dot(dmat, adj, precision=hi), dmat, precision=hi)

    xw = jnp.dot(x_ref[:], w1_ref[:], precision=hi)
    h = jnp.maximum(jnp.dot(a, xw, precision=hi) + b1_ref[:], 0.0)

    ah = jnp.dot(a, h, precision=hi)
    out_ref[:] = jnp.dot(ah, w2_ref[:], precision=hi) + b2_ref[:]


@functools.partial(jax.jit, static_argnames=("interpret",))
def kernel(x, edge_index, W1, b1, W2, b2, interpret=False):
    src = edge_index[0].astype(jnp.int32).reshape(_E, 1)
    dst = edge_index[1].astype(jnp.int32).reshape(_E, 1)
    x_pad = jnp.zeros((_NP, x.shape[1]), jnp.float32).at[:_N].set(x)

    out = pl.pallas_call(
        _gcn_tc_kernel,
        out_shape=jax.ShapeDtypeStruct((_NP, W2.shape[1]), jnp.float32),
        interpret=interpret,
    )(src, dst, x_pad, W1, b1.reshape(1, -1), W2, b2.reshape(1, -1))
    return out[:_N].reshape(_N * W2.shape[1])
